# baseline (speedup 1.0000x reference)
import numpy as np

import jax
import jax.numpy as jnp
from jax.experimental import pallas as pl
from jax.experimental.pallas import tpu as pltpu
from jax.sharding import Mesh, PartitionSpec as P

_NUM_CLASS = 1000
_TILE_M = 1024


def _cdiv(a: int, b: int) -> int:
    return (a + b - 1) // b


def _linear_kernel(x_ref, w_ref, b_ref, o_ref):
    xb = x_ref[...].astype(jnp.bfloat16)
    wb = w_ref[...].astype(jnp.bfloat16)
    acc = jnp.dot(xb, wb, preferred_element_type=jnp.float32)
    out = acc + b_ref[...]
    o_ref[...] = out[:, :_NUM_CLASS].astype(o_ref.dtype)


def _linear_pallas(x, w_t, b):
    B, D = x.shape
    Dw, Cp = w_t.shape
    tile_m = min(_TILE_M, B)
    grid = (_cdiv(B, tile_m),)
    return pl.pallas_call(
        _linear_kernel,
        out_shape=jax.ShapeDtypeStruct((B, _NUM_CLASS), x.dtype),
        grid=grid,
        in_specs=[
            pl.BlockSpec((tile_m, D), lambda i: (i, 0)),
            pl.BlockSpec((D, Cp), lambda i: (0, 0)),
            pl.BlockSpec((1, Cp), lambda i: (0, 0)),
        ],
        out_specs=pl.BlockSpec((tile_m, _NUM_CLASS), lambda i: (i, 0)),
        compiler_params=pltpu.CompilerParams(
            dimension_semantics=("arbitrary",)),
    )(x, w_t, b)


def kernel(x, w_t, b):
    B, D = x.shape
    Dw, Cp = w_t.shape
    assert D == Dw and _NUM_CLASS <= Cp

    devs = jax.devices()
    n_tc = 2 if (len(devs) >= 2 and B % (2 * _TILE_M) == 0) else 1
    if n_tc == 1:
        return _linear_pallas(x, w_t, b)

    mesh = Mesh(np.array(devs[:2]), ("tc",))
    shard = jax.shard_map(
        _linear_pallas,
        mesh=mesh,
        in_specs=(P("tc", None), P(None, None), P(None, None)),
        out_specs=P("tc", None),
        check_vma=False,
    )
    return shard(x, w_t, b)


# speedup vs baseline: 4.9765x; 4.9765x over previous
import functools

import jax
import jax.numpy as jnp
from jax.experimental import pallas as pl
from jax.experimental.pallas import tpu as pltpu

_NUM_CLASS = 1000
_TILE_M = 1024
_NBUF_IN = 3


def _pipeline_kernel(x_hbm, w_hbm, b_ref, o_hbm,
                     x_buf, o_buf, w_buf, wb_buf, in_sems, out_sems, w_sem,
                     *, n_steps, tile_m):

    def start_in(step):
        slot = step % _NBUF_IN
        pltpu.make_async_copy(
            x_hbm.at[pl.ds(step * tile_m, tile_m), :],
            x_buf.at[slot],
            in_sems.at[slot],
        ).start(priority=0)

    def wait_in(step):
        slot = step % _NBUF_IN
        pltpu.make_async_copy(
            x_buf.at[slot], x_buf.at[slot], in_sems.at[slot]).wait()

    def start_out(step):
        slot = step % 2
        pltpu.make_async_copy(
            o_buf.at[slot],
            o_hbm.at[pl.ds(step * tile_m, tile_m), :],
            out_sems.at[slot],
        ).start(priority=1)

    def wait_out(step):
        slot = step % 2
        pltpu.make_async_copy(
            o_buf.at[slot], o_buf.at[slot], out_sems.at[slot]).wait()

    pltpu.make_async_copy(w_hbm, w_buf, w_sem).start(priority=1)
    for s in range(min(_NBUF_IN, n_steps)):
        start_in(s)
    pltpu.make_async_copy(w_hbm, w_buf, w_sem).wait()
    wb_buf[...] = w_buf[...].astype(jnp.bfloat16)

    for i in range(n_steps):
        wait_in(i)
        if i >= 2:
            wait_out(i - 2)
        xb = x_buf[i % _NBUF_IN].astype(jnp.bfloat16)
        acc = jnp.dot(xb, wb_buf[...], preferred_element_type=jnp.float32)
        o_buf[i % 2] = (acc + b_ref[...])[:, :_NUM_CLASS]
        start_out(i)
        if i + _NBUF_IN < n_steps:
            start_in(i + _NBUF_IN)

    wait_out(n_steps - 2)
    wait_out(n_steps - 1)


def kernel(x, w_t, b):
    B, D = x.shape
    Dw, Cp = w_t.shape
    assert D == Dw and _NUM_CLASS <= Cp
    tile_m = min(_TILE_M, B)
    assert B % tile_m == 0
    n_steps = B // tile_m

    body = functools.partial(_pipeline_kernel, n_steps=n_steps, tile_m=tile_m)
    return pl.pallas_call(
        body,
        out_shape=jax.ShapeDtypeStruct((B, _NUM_CLASS), x.dtype),
        in_specs=[
            pl.BlockSpec(memory_space=pltpu.MemorySpace.HBM),
            pl.BlockSpec(memory_space=pltpu.MemorySpace.HBM),
            pl.BlockSpec(memory_space=pltpu.MemorySpace.VMEM),
        ],
        out_specs=pl.BlockSpec(memory_space=pltpu.MemorySpace.HBM),
        scratch_shapes=[
            pltpu.VMEM((_NBUF_IN, tile_m, D), jnp.float32),
            pltpu.VMEM((2, tile_m, _NUM_CLASS), jnp.float32),
            pltpu.VMEM((D, Cp), jnp.float32),
            pltpu.VMEM((D, Cp), jnp.bfloat16),
            pltpu.SemaphoreType.DMA((_NBUF_IN,)),
            pltpu.SemaphoreType.DMA((2,)),
            pltpu.SemaphoreType.DMA,
        ],
        compiler_params=pltpu.CompilerParams(
            vmem_limit_bytes=60 * 1024 * 1024),
    )(x, w_t, b)


# speedup vs baseline: 5.2766x; 1.0603x over previous
import jax
import jax.numpy as jnp
from jax.experimental import pallas as pl
from jax.experimental.pallas import tpu as pltpu

_NUM_CLASS = 1000
_TILE_M = 1024


def _cdiv(a: int, b: int) -> int:
    return (a + b - 1) // b


def _linear_kernel(x_ref, w_ref, b_ref, o_ref):
    xb = x_ref[...].astype(jnp.bfloat16)
    wb = w_ref[...].astype(jnp.bfloat16)
    acc = jnp.dot(xb, wb, preferred_element_type=jnp.float32)
    out = acc + b_ref[...]
    o_ref[...] = out[:, :_NUM_CLASS].astype(o_ref.dtype)


def kernel(x, w_t, b):
    B, D = x.shape
    Dw, Cp = w_t.shape
    assert D == Dw and _NUM_CLASS <= Cp

    tile_m = min(_TILE_M, B)
    grid = (_cdiv(B, tile_m),)
    return pl.pallas_call(
        _linear_kernel,
        out_shape=jax.ShapeDtypeStruct((B, _NUM_CLASS), x.dtype),
        grid=grid,
        in_specs=[
            pl.BlockSpec((tile_m, D), lambda i: (i, 0)),
            pl.BlockSpec((D, Cp), lambda i: (0, 0)),
            pl.BlockSpec((1, Cp), lambda i: (0, 0)),
        ],
        out_specs=pl.BlockSpec((tile_m, _NUM_CLASS), lambda i: (i, 0)),
        compiler_params=pltpu.CompilerParams(
            dimension_semantics=("parallel",)),
    )(x, w_t, b)
